# baseline (speedup 1.0000x reference)
import functools

import jax
import jax.numpy as jnp
from jax import lax
from jax.experimental import pallas as pl
from jax.experimental.pallas import tpu as pltpu
from jax.experimental.pallas import tpu_sc as plsc

N = 10000
E = 320000
IN_F = 128
HID = 256
H1 = 8
DH1 = 32
C = 64

NC = 2
NS = 16
L = 16

BN = 400
NB = N // BN
K = 80
EPT = E // NS
NCHUNK = EPT // K
NPT = 640
NPT_LAST = N - 15 * NPT

f32 = jnp.float32


def _tc1_body(x_ref, w1_ref, a1s_ref, a1d_ref, h_ref, ss_ref, sd_ref):
    h = jnp.dot(x_ref[...], w1_ref[...], preferred_element_type=f32)
    h_ref[0] = h
    ss_ref[0] = jnp.dot(h, a1s_ref[0], preferred_element_type=f32)
    sd_ref[0] = jnp.dot(h, a1d_ref[0], preferred_element_type=f32)


def _tc1(x, W1, A1s, A1d):
    return pl.pallas_call(
        _tc1_body,
        grid=(NC, NB),
        in_specs=[
            pl.BlockSpec((BN, IN_F), lambda c, i: (i, 0)),
            pl.BlockSpec((IN_F, IN_F), lambda c, i: (0, c)),
            pl.BlockSpec((1, IN_F, 16), lambda c, i: (c, 0, 0)),
            pl.BlockSpec((1, IN_F, 16), lambda c, i: (c, 0, 0)),
        ],
        out_specs=[
            pl.BlockSpec((1, BN, IN_F), lambda c, i: (c, i, 0)),
            pl.BlockSpec((1, BN, 16), lambda c, i: (c, i, 0)),
            pl.BlockSpec((1, BN, 16), lambda c, i: (c, i, 0)),
        ],
        out_shape=[
            jax.ShapeDtypeStruct((NC, N, IN_F), f32),
            jax.ShapeDtypeStruct((NC, N, 16), f32),
            jax.ShapeDtypeStruct((NC, N, 16), f32),
        ],
    )(x, W1, A1s, A1d)


def _tc2_body(num_ref, den_ref, erep_ref, w2_ref, a2s_ref, a2d_ref,
              h2_ref, ss_ref, sd_ref):
    c = pl.program_id(0)
    d0 = jnp.dot(den_ref[0], erep_ref[...], preferred_element_type=f32)
    d1 = jnp.dot(den_ref[1], erep_ref[...], preferred_element_type=f32)
    a0 = num_ref[0] / (d0 + 1e-16)
    a1 = num_ref[1] / (d1 + 1e-16)
    z = jnp.concatenate([a0, a1], axis=1)
    z = jnp.where(z > 0, z, jnp.exp(z) - 1.0)
    h2 = jnp.dot(z, w2_ref[...], preferred_element_type=f32)
    ss_ref[0] = jnp.dot(h2, a2s_ref[...], preferred_element_type=f32)
    sd_ref[0] = jnp.dot(h2, a2d_ref[...], preferred_element_type=f32)
    h2_ref[0] = jnp.where(c == 0, h2[:, :32], h2[:, 32:])


def _tc2(num1, den1, Erep, W2, A2s, A2d):
    return pl.pallas_call(
        _tc2_body,
        grid=(NC, NB),
        in_specs=[
            pl.BlockSpec((NC, BN, IN_F), lambda c, i: (0, i, 0)),
            pl.BlockSpec((NC, BN, 16), lambda c, i: (0, i, 0)),
            pl.BlockSpec((16, HID // 2), lambda c, i: (0, 0)),
            pl.BlockSpec((HID, C), lambda c, i: (0, 0)),
            pl.BlockSpec((C, 16), lambda c, i: (0, 0)),
            pl.BlockSpec((C, 16), lambda c, i: (0, 0)),
        ],
        out_specs=[
            pl.BlockSpec((1, BN, 32), lambda c, i: (c, i, 0)),
            pl.BlockSpec((1, BN, 16), lambda c, i: (c, i, 0)),
            pl.BlockSpec((1, BN, 16), lambda c, i: (c, i, 0)),
        ],
        out_shape=[
            jax.ShapeDtypeStruct((NC, N, 32), f32),
            jax.ShapeDtypeStruct((NC, N, 16), f32),
            jax.ShapeDtypeStruct((NC, N, 16), f32),
        ],
    )(num1, den1, Erep, W2, A2s, A2d)


def _tc3_body(num_ref, den_ref, e2_ref, out_ref):
    d = jnp.dot(den_ref[0], e2_ref[...], preferred_element_type=f32)
    z = jnp.concatenate([num_ref[0], num_ref[1]], axis=1) / (d + 1e-16)
    z = jnp.where(z > 0, z, jnp.exp(z) - 1.0)
    m = jnp.max(z, axis=1, keepdims=True)
    t = z - m
    out_ref[...] = t - jnp.log(jnp.sum(jnp.exp(t), axis=1, keepdims=True))


def _tc3(num2, den2, E2):
    return pl.pallas_call(
        _tc3_body,
        grid=(NB,),
        in_specs=[
            pl.BlockSpec((NC, BN, 32), lambda i: (0, i, 0)),
            pl.BlockSpec((NC, BN, 16), lambda i: (0, i, 0)),
            pl.BlockSpec((16, C), lambda i: (0, 0)),
        ],
        out_specs=pl.BlockSpec((BN, C), lambda i: (i, 0)),
        out_shape=jax.ShapeDtypeStruct((N, C), f32),
    )(num2, den2, E2)


def _make_sc_edge(F, U):
    n_iter = NCHUNK // U
    n_pairs = F // 32
    mesh = plsc.VectorSubcoreMesh(core_axis_name="c", subcore_axis_name="s")

    def body(h_tab, ss_tab, sd_tab, idxtab, num_out, den_out,
             idxb, rows, ssb, sdb, accn, accd, *sems):
        gsem = sems[:U]
        ssem = sems[U:]
        cid = lax.axis_index("c")
        sid = lax.axis_index("s")
        chunk_base = (cid * NS + sid) * NCHUNK

        zero16 = jnp.zeros((L,), f32)

        def zrow(k, _):
            def zcol(g, _):
                rows[0, k, pl.ds(g * L, L)] = zero16
                return 0
            lax.fori_loop(0, F // L, zcol, 0)
            ssb[0, k, :] = zero16
            return 0
        lax.fori_loop(0, K, zrow, 0)

        nz = jnp.where(sid < 15, NPT // K, NPT_LAST // K)

        def zacc(t, _):
            off = sid * NPT + t * K
            pltpu.sync_copy(rows.at[0], accn.at[pl.ds(off, K)])
            pltpu.sync_copy(ssb.at[0], accd.at[pl.ds(off, K)])
            return 0
        lax.fori_loop(0, nz, zacc, 0)
        plsc.subcore_barrier()

        def giter(t, _):
            g = []
            for p in range(U):
                ch = chunk_base + t * U + p
                pltpu.sync_copy(idxtab.at[ch], idxb.at[p])
                g.append((
                    pltpu.async_copy(h_tab.at[idxb.at[p, 0]], rows.at[p],
                                     gsem[p]),
                    pltpu.async_copy(ss_tab.at[idxb.at[p, 0]], ssb.at[p],
                                     gsem[p]),
                    pltpu.async_copy(sd_tab.at[idxb.at[p, 1]], sdb.at[p],
                                     gsem[p]),
                ))
            s = []
            for p in range(U):
                for d in g[p]:
                    d.wait()

                def scale(k, _):
                    e = ssb[p, k, :] + sdb[p, k, :]
                    w = jnp.exp(jnp.maximum(e, 0.2 * e))
                    ssb[p, k, :] = w
                    for q in range(n_pairs):
                        wv = w.at[jnp.full((L,), q, jnp.int32)].get(
                            mode="promise_in_bounds")
                        for half in range(2):
                            c0 = q * 32 + half * L
                            rows[p, k, pl.ds(c0, L)] = (
                                rows[p, k, pl.ds(c0, L)] * wv)
                    return 0
                lax.fori_loop(0, K, scale, 0, unroll=8)
                s.append(pltpu.async_copy(rows.at[p], accn.at[idxb.at[p, 2]],
                                          ssem[p], add=True))
                s.append(pltpu.async_copy(ssb.at[p], accd.at[idxb.at[p, 2]],
                                          ssem[p], add=True))
            for d in s:
                d.wait()
            return 0
        lax.fori_loop(0, n_iter, giter, 0)
        plsc.subcore_barrier()

        off = sid * NPT

        @pl.when(sid < 15)
        def _():
            pltpu.sync_copy(accn.at[pl.ds(off, NPT)],
                            num_out.at[cid, pl.ds(off, NPT)])
            pltpu.sync_copy(accd.at[pl.ds(off, NPT)],
                            den_out.at[cid, pl.ds(off, NPT)])

        @pl.when(sid == 15)
        def _():
            pltpu.sync_copy(accn.at[pl.ds(off, NPT_LAST)],
                            num_out.at[cid, pl.ds(off, NPT_LAST)])
            pltpu.sync_copy(accd.at[pl.ds(off, NPT_LAST)],
                            den_out.at[cid, pl.ds(off, NPT_LAST)])

    return pl.kernel(
        body,
        out_type=(
            jax.ShapeDtypeStruct((NC, N, F), f32),
            jax.ShapeDtypeStruct((NC, N, 16), f32),
        ),
        mesh=mesh,
        compiler_params=pltpu.CompilerParams(use_tc_tiling_on_sc=False),
        scratch_types=[
            pltpu.VMEM((U, 3, K), jnp.int32),
            pltpu.VMEM((U, K, F), f32),
            pltpu.VMEM((U, K, 16), f32),
            pltpu.VMEM((U, K, 16), f32),
            pltpu.VMEM_SHARED((N, F), f32),
            pltpu.VMEM_SHARED((N, 16), f32),
        ] + [pltpu.SemaphoreType.DMA] * (2 * U),
    )


_sc_edge_128 = _make_sc_edge(IN_F, 2)
_sc_edge_32 = _make_sc_edge(32, 5)


def kernel(x, edge_index, W1, a1_src, a1_dst, W2, a2_src, a2_dst):
    src = edge_index[0]
    dst = edge_index[1]
    base = jnp.stack([src, dst, dst])
    idx_parts = []
    for c in range(NC):
        off = jnp.array([c * N, c * N, 0], jnp.int32)[:, None]
        t = (base + off).reshape(3, NS, NCHUNK, K).transpose(1, 2, 0, 3)
        idx_parts.append(t)
    idxtab = jnp.stack(idx_parts).reshape(NC * NS * NCHUNK, 3, K)

    eye = jnp.eye(16, dtype=f32)
    A1s = (a1_src.reshape(2, 4, DH1)[:, :, :, None]
           * eye[None, :4, None, :]).reshape(2, IN_F, 16)
    A1d = (a1_dst.reshape(2, 4, DH1)[:, :, :, None]
           * eye[None, :4, None, :]).reshape(2, IN_F, 16)
    A2s = jnp.zeros((C, 16), f32).at[:, 0].set(a2_src[0])
    A2d = jnp.zeros((C, 16), f32).at[:, 0].set(a2_dst[0])
    Erep = jnp.repeat(eye[:, :4], DH1, axis=1)
    E2 = jnp.zeros((16, C), f32).at[0, :].set(1.0)

    h1, ss1, sd1 = _tc1(x, W1, A1s, A1d)
    num1, den1 = _sc_edge_128(
        h1.reshape(NC * N, IN_F), ss1.reshape(NC * N, 16),
        sd1.reshape(NC * N, 16), idxtab)
    h2, ss2, sd2 = _tc2(num1, den1, Erep, W2, A2s, A2d)
    num2, den2 = _sc_edge_32(
        h2.reshape(NC * N, 32), ss2.reshape(NC * N, 16),
        sd2.reshape(NC * N, 16), idxtab)
    return _tc3(num2, den2, E2)


# speedup vs baseline: 1.2359x; 1.2359x over previous
import functools

import jax
import jax.numpy as jnp
from jax import lax
from jax.experimental import pallas as pl
from jax.experimental.pallas import tpu as pltpu
from jax.experimental.pallas import tpu_sc as plsc

N = 10000
E = 320000
IN_F = 128
HID = 256
H1 = 8
DH1 = 32
C = 64

NC = 2
NS = 16
L = 16

BN = 400
NB = N // BN
K = 80
EPT = E // NS
NCHUNK = EPT // K
NPT = 640
NPT_LAST = N - 15 * NPT

f32 = jnp.float32


def _tc1_body(x_ref, w1_ref, a1s_ref, a1d_ref, h_ref, ss_ref, sd_ref):
    h = jnp.dot(x_ref[...], w1_ref[...], preferred_element_type=f32)
    h_ref[0] = h
    ss_ref[0] = jnp.dot(h, a1s_ref[0], preferred_element_type=f32)
    sd_ref[0] = jnp.dot(h, a1d_ref[0], preferred_element_type=f32)


def _tc1(x, W1, A1s, A1d):
    return pl.pallas_call(
        _tc1_body,
        grid=(NC, NB),
        in_specs=[
            pl.BlockSpec((BN, IN_F), lambda c, i: (i, 0)),
            pl.BlockSpec((IN_F, IN_F), lambda c, i: (0, c)),
            pl.BlockSpec((1, IN_F, 16), lambda c, i: (c, 0, 0)),
            pl.BlockSpec((1, IN_F, 16), lambda c, i: (c, 0, 0)),
        ],
        out_specs=[
            pl.BlockSpec((1, BN, IN_F), lambda c, i: (c, i, 0)),
            pl.BlockSpec((1, BN, 16), lambda c, i: (c, i, 0)),
            pl.BlockSpec((1, BN, 16), lambda c, i: (c, i, 0)),
        ],
        out_shape=[
            jax.ShapeDtypeStruct((NC, N, IN_F), f32),
            jax.ShapeDtypeStruct((NC, N, 16), f32),
            jax.ShapeDtypeStruct((NC, N, 16), f32),
        ],
    )(x, W1, A1s, A1d)


def _tc2_body(num_ref, den_ref, erep_ref, w2_ref, a2s_ref, a2d_ref,
              h2_ref, ss_ref, sd_ref):
    c = pl.program_id(0)
    d0 = jnp.dot(den_ref[0], erep_ref[...], preferred_element_type=f32)
    d1 = jnp.dot(den_ref[1], erep_ref[...], preferred_element_type=f32)
    a0 = num_ref[0] / (d0 + 1e-16)
    a1 = num_ref[1] / (d1 + 1e-16)
    z = jnp.concatenate([a0, a1], axis=1)
    z = jnp.where(z > 0, z, jnp.exp(z) - 1.0)
    h2 = jnp.dot(z, w2_ref[...], preferred_element_type=f32)
    ss_ref[0] = jnp.dot(h2, a2s_ref[...], preferred_element_type=f32)
    sd_ref[0] = jnp.dot(h2, a2d_ref[...], preferred_element_type=f32)
    h2_ref[0] = jnp.where(c == 0, h2[:, :32], h2[:, 32:])


def _tc2(num1, den1, Erep, W2, A2s, A2d):
    return pl.pallas_call(
        _tc2_body,
        grid=(NC, NB),
        in_specs=[
            pl.BlockSpec((NC, BN, IN_F), lambda c, i: (0, i, 0)),
            pl.BlockSpec((NC, BN, 16), lambda c, i: (0, i, 0)),
            pl.BlockSpec((16, HID // 2), lambda c, i: (0, 0)),
            pl.BlockSpec((HID, C), lambda c, i: (0, 0)),
            pl.BlockSpec((C, 16), lambda c, i: (0, 0)),
            pl.BlockSpec((C, 16), lambda c, i: (0, 0)),
        ],
        out_specs=[
            pl.BlockSpec((1, BN, 32), lambda c, i: (c, i, 0)),
            pl.BlockSpec((1, BN, 16), lambda c, i: (c, i, 0)),
            pl.BlockSpec((1, BN, 16), lambda c, i: (c, i, 0)),
        ],
        out_shape=[
            jax.ShapeDtypeStruct((NC, N, 32), f32),
            jax.ShapeDtypeStruct((NC, N, 16), f32),
            jax.ShapeDtypeStruct((NC, N, 16), f32),
        ],
    )(num1, den1, Erep, W2, A2s, A2d)


def _tc3_body(num_ref, den_ref, e2_ref, out_ref):
    d = jnp.dot(den_ref[0], e2_ref[...], preferred_element_type=f32)
    z = jnp.concatenate([num_ref[0], num_ref[1]], axis=1) / (d + 1e-16)
    z = jnp.where(z > 0, z, jnp.exp(z) - 1.0)
    m = jnp.max(z, axis=1, keepdims=True)
    t = z - m
    out_ref[...] = t - jnp.log(jnp.sum(jnp.exp(t), axis=1, keepdims=True))


def _tc3(num2, den2, E2):
    return pl.pallas_call(
        _tc3_body,
        grid=(NB,),
        in_specs=[
            pl.BlockSpec((NC, BN, 32), lambda i: (0, i, 0)),
            pl.BlockSpec((NC, BN, 16), lambda i: (0, i, 0)),
            pl.BlockSpec((16, C), lambda i: (0, 0)),
        ],
        out_specs=pl.BlockSpec((BN, C), lambda i: (i, 0)),
        out_shape=jax.ShapeDtypeStruct((N, C), f32),
    )(num2, den2, E2)


def _make_sc_edge(F, U):
    n_iter = NCHUNK // U
    n_pairs = F // 32
    mesh = plsc.VectorSubcoreMesh(core_axis_name="c", subcore_axis_name="s")

    def body(h_tab, ss_tab, sd_tab, idxtab, num_out, den_out,
             idxb, rows, ssb, sdb, accn, accd, *sems):
        gsem = sems[:U]
        ssem = sems[U:]
        cid = lax.axis_index("c")
        sid = lax.axis_index("s")
        iter_base = (cid * NS + sid) * n_iter

        zero16 = jnp.zeros((L,), f32)

        def zrow(k, _):
            def zcol(g, _):
                rows[0, k, pl.ds(g * L, L)] = zero16
                return 0
            lax.fori_loop(0, F // L, zcol, 0)
            ssb[0, k, :] = zero16
            return 0
        lax.fori_loop(0, K, zrow, 0)

        nz = jnp.where(sid < 15, NPT // K, NPT_LAST // K)

        def zacc(t, _):
            off = sid * NPT + t * K
            pltpu.sync_copy(rows.at[0], accn.at[pl.ds(off, K)])
            pltpu.sync_copy(ssb.at[0], accd.at[pl.ds(off, K)])
            return 0
        lax.fori_loop(0, nz, zacc, 0)
        plsc.subcore_barrier()

        def giter(t, _):
            pltpu.sync_copy(idxtab.at[iter_base + t], idxb)
            g = []
            for p in range(U):
                g.append((
                    pltpu.async_copy(h_tab.at[idxb.at[3 * p]], rows.at[p],
                                     gsem[p]),
                    pltpu.async_copy(ss_tab.at[idxb.at[3 * p]], ssb.at[p],
                                     gsem[p]),
                    pltpu.async_copy(sd_tab.at[idxb.at[3 * p + 1]], sdb.at[p],
                                     gsem[p]),
                ))
            s = []
            for p in range(U):
                for d in g[p]:
                    d.wait()

                def scale(k, _):
                    e = ssb[p, k, :] + sdb[p, k, :]
                    w = jnp.exp(jnp.maximum(e, 0.2 * e))
                    ssb[p, k, :] = w
                    for q in range(n_pairs):
                        wv = w.at[jnp.full((L,), q, jnp.int32)].get(
                            mode="promise_in_bounds")
                        for half in range(2):
                            c0 = q * 32 + half * L
                            rows[p, k, pl.ds(c0, L)] = (
                                rows[p, k, pl.ds(c0, L)] * wv)
                    return 0
                lax.fori_loop(0, K, scale, 0)
                s.append(pltpu.async_copy(rows.at[p],
                                          accn.at[idxb.at[3 * p + 2]],
                                          ssem[p], add=True))
                s.append(pltpu.async_copy(ssb.at[p],
                                          accd.at[idxb.at[3 * p + 2]],
                                          ssem[p], add=True))
            for d in s:
                d.wait()
            return 0
        lax.fori_loop(0, n_iter, giter, 0)
        plsc.subcore_barrier()

        off = sid * NPT

        @pl.when(sid < 15)
        def _():
            pltpu.sync_copy(accn.at[pl.ds(off, NPT)],
                            num_out.at[cid, pl.ds(off, NPT)])
            pltpu.sync_copy(accd.at[pl.ds(off, NPT)],
                            den_out.at[cid, pl.ds(off, NPT)])

        @pl.when(sid == 15)
        def _():
            pltpu.sync_copy(accn.at[pl.ds(off, NPT_LAST)],
                            num_out.at[cid, pl.ds(off, NPT_LAST)])
            pltpu.sync_copy(accd.at[pl.ds(off, NPT_LAST)],
                            den_out.at[cid, pl.ds(off, NPT_LAST)])

    return pl.kernel(
        body,
        out_type=(
            jax.ShapeDtypeStruct((NC, N, F), f32),
            jax.ShapeDtypeStruct((NC, N, 16), f32),
        ),
        mesh=mesh,
        compiler_params=pltpu.CompilerParams(use_tc_tiling_on_sc=False),
        scratch_types=[
            pltpu.VMEM((3 * U, K), jnp.int32),
            pltpu.VMEM((U, K, F), f32),
            pltpu.VMEM((U, K, 16), f32),
            pltpu.VMEM((U, K, 16), f32),
            pltpu.VMEM_SHARED((N, F), f32),
            pltpu.VMEM_SHARED((N, 16), f32),
        ] + [pltpu.SemaphoreType.DMA] * (2 * U),
    )


_sc_edge_128 = _make_sc_edge(IN_F, 2)
_sc_edge_32 = _make_sc_edge(32, 5)


def kernel(x, edge_index, W1, a1_src, a1_dst, W2, a2_src, a2_dst):
    src = edge_index[0]
    dst = edge_index[1]
    base = jnp.stack([src, dst, dst])

    def build_idxtab(u):
        n_it = NCHUNK // u
        parts = []
        for c in range(NC):
            off = jnp.array([c * N, c * N, 0], jnp.int32)[:, None]
            t = (base + off).reshape(3, NS, n_it, u, K)
            parts.append(t.transpose(1, 2, 3, 0, 4))
        return jnp.stack(parts).reshape(NC * NS * n_it, 3 * u, K)

    idxtab1 = build_idxtab(2)
    idxtab2 = build_idxtab(5)

    eye = jnp.eye(16, dtype=f32)
    A1s = (a1_src.reshape(2, 4, DH1)[:, :, :, None]
           * eye[None, :4, None, :]).reshape(2, IN_F, 16)
    A1d = (a1_dst.reshape(2, 4, DH1)[:, :, :, None]
           * eye[None, :4, None, :]).reshape(2, IN_F, 16)
    A2s = jnp.zeros((C, 16), f32).at[:, 0].set(a2_src[0])
    A2d = jnp.zeros((C, 16), f32).at[:, 0].set(a2_dst[0])
    Erep = jnp.repeat(eye[:, :4], DH1, axis=1)
    E2 = jnp.zeros((16, C), f32).at[0, :].set(1.0)

    h1, ss1, sd1 = _tc1(x, W1, A1s, A1d)
    num1, den1 = _sc_edge_128(
        h1.reshape(NC * N, IN_F), ss1.reshape(NC * N, 16),
        sd1.reshape(NC * N, 16), idxtab1)
    h2, ss2, sd2 = _tc2(num1, den1, Erep, W2, A2s, A2d)
    num2, den2 = _sc_edge_32(
        h2.reshape(NC * N, 32), ss2.reshape(NC * N, 16),
        sd2.reshape(NC * N, 16), idxtab2)
    return _tc3(num2, den2, E2)


# speedup vs baseline: 1.3043x; 1.0554x over previous
import functools

import jax
import jax.numpy as jnp
from jax import lax
from jax.experimental import pallas as pl
from jax.experimental.pallas import tpu as pltpu
from jax.experimental.pallas import tpu_sc as plsc

N = 10000
E = 320000
IN_F = 128
HID = 256
H1 = 8
DH1 = 32
C = 64

NC = 2
NS = 16
L = 16

BN = 400
NB = N // BN
K = 80
EPT = E // NS
NCHUNK = EPT // K
NPT = 640
NPT_LAST = N - 15 * NPT

f32 = jnp.float32


def _tc1_body(x_ref, w1_ref, a1s_ref, a1d_ref, h_ref, ss_ref, sd_ref):
    h = jnp.dot(x_ref[...], w1_ref[...], preferred_element_type=f32)
    h_ref[0] = h
    ss_ref[0] = jnp.dot(h, a1s_ref[0], preferred_element_type=f32)
    sd_ref[0] = jnp.dot(h, a1d_ref[0], preferred_element_type=f32)


def _tc1(x, W1, A1s, A1d):
    return pl.pallas_call(
        _tc1_body,
        grid=(NC, NB),
        in_specs=[
            pl.BlockSpec((BN, IN_F), lambda c, i: (i, 0)),
            pl.BlockSpec((IN_F, IN_F), lambda c, i: (0, c)),
            pl.BlockSpec((1, IN_F, 16), lambda c, i: (c, 0, 0)),
            pl.BlockSpec((1, IN_F, 16), lambda c, i: (c, 0, 0)),
        ],
        out_specs=[
            pl.BlockSpec((1, BN, IN_F), lambda c, i: (c, i, 0)),
            pl.BlockSpec((1, BN, 16), lambda c, i: (c, i, 0)),
            pl.BlockSpec((1, BN, 16), lambda c, i: (c, i, 0)),
        ],
        out_shape=[
            jax.ShapeDtypeStruct((NC, N, IN_F), f32),
            jax.ShapeDtypeStruct((NC, N, 16), f32),
            jax.ShapeDtypeStruct((NC, N, 16), f32),
        ],
    )(x, W1, A1s, A1d)


def _tc2_body(num_ref, den_ref, erep_ref, w2_ref, a2s_ref, a2d_ref,
              h2_ref, ss_ref, sd_ref):
    c = pl.program_id(0)
    d0 = jnp.dot(den_ref[0], erep_ref[...], preferred_element_type=f32)
    d1 = jnp.dot(den_ref[1], erep_ref[...], preferred_element_type=f32)
    a0 = num_ref[0] / (d0 + 1e-16)
    a1 = num_ref[1] / (d1 + 1e-16)
    z = jnp.concatenate([a0, a1], axis=1)
    z = jnp.where(z > 0, z, jnp.exp(z) - 1.0)
    h2 = jnp.dot(z, w2_ref[...], preferred_element_type=f32)
    ss_ref[0] = jnp.dot(h2, a2s_ref[...], preferred_element_type=f32)
    sd_ref[0] = jnp.dot(h2, a2d_ref[...], preferred_element_type=f32)
    h2_ref[0] = jnp.where(c == 0, h2[:, :32], h2[:, 32:])


def _tc2(num1, den1, Erep, W2, A2s, A2d):
    return pl.pallas_call(
        _tc2_body,
        grid=(NC, NB),
        in_specs=[
            pl.BlockSpec((NC, BN, IN_F), lambda c, i: (0, i, 0)),
            pl.BlockSpec((NC, BN, 16), lambda c, i: (0, i, 0)),
            pl.BlockSpec((16, HID // 2), lambda c, i: (0, 0)),
            pl.BlockSpec((HID, C), lambda c, i: (0, 0)),
            pl.BlockSpec((C, 16), lambda c, i: (0, 0)),
            pl.BlockSpec((C, 16), lambda c, i: (0, 0)),
        ],
        out_specs=[
            pl.BlockSpec((1, BN, 32), lambda c, i: (c, i, 0)),
            pl.BlockSpec((1, BN, 16), lambda c, i: (c, i, 0)),
            pl.BlockSpec((1, BN, 16), lambda c, i: (c, i, 0)),
        ],
        out_shape=[
            jax.ShapeDtypeStruct((NC, N, 32), f32),
            jax.ShapeDtypeStruct((NC, N, 16), f32),
            jax.ShapeDtypeStruct((NC, N, 16), f32),
        ],
    )(num1, den1, Erep, W2, A2s, A2d)


def _tc3_body(num_ref, den_ref, e2_ref, out_ref):
    d = jnp.dot(den_ref[0], e2_ref[...], preferred_element_type=f32)
    z = jnp.concatenate([num_ref[0], num_ref[1]], axis=1) / (d + 1e-16)
    z = jnp.where(z > 0, z, jnp.exp(z) - 1.0)
    m = jnp.max(z, axis=1, keepdims=True)
    t = z - m
    out_ref[...] = t - jnp.log(jnp.sum(jnp.exp(t), axis=1, keepdims=True))


def _tc3(num2, den2, E2):
    return pl.pallas_call(
        _tc3_body,
        grid=(NB,),
        in_specs=[
            pl.BlockSpec((NC, BN, 32), lambda i: (0, i, 0)),
            pl.BlockSpec((NC, BN, 16), lambda i: (0, i, 0)),
            pl.BlockSpec((16, C), lambda i: (0, 0)),
        ],
        out_specs=pl.BlockSpec((BN, C), lambda i: (i, 0)),
        out_shape=jax.ShapeDtypeStruct((N, C), f32),
    )(num2, den2, E2)


def _make_sc_edge(F, U):
    n_iter = NCHUNK // U
    n_pairs = F // 32
    mesh = plsc.VectorSubcoreMesh(core_axis_name="c", subcore_axis_name="s")

    def body(h_tab, ss_tab, sd_tab, idxtab, num_out, den_out,
             idxb, rows, ssb, sdb, accn, accd, *sems):
        gsem = sems[:U]
        ssem = sems[U:]
        cid = lax.axis_index("c")
        sid = lax.axis_index("s")
        chunk_base = (cid * NS + sid) * NCHUNK

        zero16 = jnp.zeros((L,), f32)

        def zrow(k, _):
            def zcol(g, _):
                rows[0, k, pl.ds(g * L, L)] = zero16
                return 0
            lax.fori_loop(0, F // L, zcol, 0)
            ssb[0, k, :] = zero16
            return 0
        lax.fori_loop(0, K, zrow, 0)

        nz = jnp.where(sid < 15, NPT // K, NPT_LAST // K)

        def zacc(t, _):
            off = sid * NPT + t * K
            pltpu.sync_copy(rows.at[0], accn.at[pl.ds(off, K)])
            pltpu.sync_copy(ssb.at[0], accd.at[pl.ds(off, K)])
            return 0
        lax.fori_loop(0, nz, zacc, 0)
        plsc.subcore_barrier()

        def giter(t, _):
            g = []
            for p in range(U):
                ch = chunk_base + t * U + p
                pltpu.sync_copy(idxtab.at[ch], idxb.at[p])
                g.append((
                    pltpu.async_copy(h_tab.at[idxb.at[p, 0]], rows.at[p],
                                     gsem[p]),
                    pltpu.async_copy(ss_tab.at[idxb.at[p, 0]], ssb.at[p],
                                     gsem[p]),
                    pltpu.async_copy(sd_tab.at[idxb.at[p, 1]], sdb.at[p],
                                     gsem[p]),
                ))
            s = []
            for p in range(U):
                for d in g[p]:
                    d.wait()

                def scale(k, _):
                    e = ssb[p, k, :] + sdb[p, k, :]
                    w = jnp.exp(jnp.maximum(e, 0.2 * e))
                    ssb[p, k, :] = w
                    for q in range(n_pairs):
                        wv = w.at[jnp.full((L,), q, jnp.int32)].get(
                            mode="promise_in_bounds")
                        for half in range(2):
                            c0 = q * 32 + half * L
                            rows[p, k, pl.ds(c0, L)] = (
                                rows[p, k, pl.ds(c0, L)] * wv)
                    return 0
                lax.fori_loop(0, K, scale, 0)
                s.append(pltpu.async_copy(rows.at[p], accn.at[idxb.at[p, 2]],
                                          ssem[p], add=True))
                s.append(pltpu.async_copy(ssb.at[p], accd.at[idxb.at[p, 2]],
                                          ssem[p], add=True))
            for d in s:
                d.wait()
            return 0
        lax.fori_loop(0, n_iter, giter, 0)
        plsc.subcore_barrier()

        off = sid * NPT

        @pl.when(sid < 15)
        def _():
            pltpu.sync_copy(accn.at[pl.ds(off, NPT)],
                            num_out.at[cid, pl.ds(off, NPT)])
            pltpu.sync_copy(accd.at[pl.ds(off, NPT)],
                            den_out.at[cid, pl.ds(off, NPT)])

        @pl.when(sid == 15)
        def _():
            pltpu.sync_copy(accn.at[pl.ds(off, NPT_LAST)],
                            num_out.at[cid, pl.ds(off, NPT_LAST)])
            pltpu.sync_copy(accd.at[pl.ds(off, NPT_LAST)],
                            den_out.at[cid, pl.ds(off, NPT_LAST)])

    return pl.kernel(
        body,
        out_type=(
            jax.ShapeDtypeStruct((NC, N, F), f32),
            jax.ShapeDtypeStruct((NC, N, 16), f32),
        ),
        mesh=mesh,
        compiler_params=pltpu.CompilerParams(use_tc_tiling_on_sc=False),
        scratch_types=[
            pltpu.VMEM((U, 3, K), jnp.int32),
            pltpu.VMEM((U, K, F), f32),
            pltpu.VMEM((U, K, 16), f32),
            pltpu.VMEM((U, K, 16), f32),
            pltpu.VMEM_SHARED((N, F), f32),
            pltpu.VMEM_SHARED((N, 16), f32),
        ] + [pltpu.SemaphoreType.DMA] * (2 * U),
    )


_sc_edge_128 = _make_sc_edge(IN_F, 2)
_sc_edge_32 = _make_sc_edge(32, 5)


def kernel(x, edge_index, W1, a1_src, a1_dst, W2, a2_src, a2_dst):
    src = edge_index[0]
    dst = edge_index[1]
    base = jnp.stack([src, dst, dst])
    idx_parts = []
    for c in range(NC):
        off = jnp.array([c * N, c * N, 0], jnp.int32)[:, None]
        t = (base + off).reshape(3, NS, NCHUNK, K).transpose(1, 2, 0, 3)
        idx_parts.append(t)
    idxtab = jnp.stack(idx_parts).reshape(NC * NS * NCHUNK, 3, K)

    eye = jnp.eye(16, dtype=f32)
    A1s = (a1_src.reshape(2, 4, DH1)[:, :, :, None]
           * eye[None, :4, None, :]).reshape(2, IN_F, 16)
    A1d = (a1_dst.reshape(2, 4, DH1)[:, :, :, None]
           * eye[None, :4, None, :]).reshape(2, IN_F, 16)
    A2s = jnp.zeros((C, 16), f32).at[:, 0].set(a2_src[0])
    A2d = jnp.zeros((C, 16), f32).at[:, 0].set(a2_dst[0])
    Erep = jnp.repeat(eye[:, :4], DH1, axis=1)
    E2 = jnp.zeros((16, C), f32).at[0, :].set(1.0)

    h1, ss1, sd1 = _tc1(x, W1, A1s, A1d)
    num1, den1 = _sc_edge_128(
        h1.reshape(NC * N, IN_F), ss1.reshape(NC * N, 16),
        sd1.reshape(NC * N, 16), idxtab)
    h2, ss2, sd2 = _tc2(num1, den1, Erep, W2, A2s, A2d)
    num2, den2 = _sc_edge_32(
        h2.reshape(NC * N, 32), ss2.reshape(NC * N, 16),
        sd2.reshape(NC * N, 16), idxtab)
    return _tc3(num2, den2, E2)


# speedup vs baseline: 1.3066x; 1.0018x over previous
import functools

import jax
import jax.numpy as jnp
from jax import lax
from jax.experimental import pallas as pl
from jax.experimental.pallas import tpu as pltpu
from jax.experimental.pallas import tpu_sc as plsc

N = 10000
E = 320000
IN_F = 128
HID = 256
H1 = 8
DH1 = 32
C = 64

NC = 2
NS = 16
L = 16

BN = 400
NB = N // BN
K = 80
EPT = E // NS
NCHUNK = EPT // K
NPT = 640
NPT_LAST = N - 15 * NPT

f32 = jnp.float32


def _tc1_body(x_ref, w1_ref, a1s_ref, a1d_ref, h_ref, ss_ref, sd_ref):
    h = jnp.dot(x_ref[...], w1_ref[...], preferred_element_type=f32)
    h_ref[0] = h
    ss_ref[0] = jnp.dot(h, a1s_ref[0], preferred_element_type=f32)
    sd_ref[0] = jnp.dot(h, a1d_ref[0], preferred_element_type=f32)


def _tc1(x, W1, A1s, A1d):
    return pl.pallas_call(
        _tc1_body,
        grid=(NC, NB),
        in_specs=[
            pl.BlockSpec((BN, IN_F), lambda c, i: (i, 0)),
            pl.BlockSpec((IN_F, IN_F), lambda c, i: (0, c)),
            pl.BlockSpec((1, IN_F, 16), lambda c, i: (c, 0, 0)),
            pl.BlockSpec((1, IN_F, 16), lambda c, i: (c, 0, 0)),
        ],
        out_specs=[
            pl.BlockSpec((1, BN, IN_F), lambda c, i: (c, i, 0)),
            pl.BlockSpec((1, BN, 16), lambda c, i: (c, i, 0)),
            pl.BlockSpec((1, BN, 16), lambda c, i: (c, i, 0)),
        ],
        out_shape=[
            jax.ShapeDtypeStruct((NC, N, IN_F), f32),
            jax.ShapeDtypeStruct((NC, N, 16), f32),
            jax.ShapeDtypeStruct((NC, N, 16), f32),
        ],
    )(x, W1, A1s, A1d)


def _tc2_body(num_ref, den_ref, erep_ref, w2_ref, a2s_ref, a2d_ref,
              h2_ref, ss_ref, sd_ref):
    c = pl.program_id(0)
    d0 = jnp.dot(den_ref[0], erep_ref[...], preferred_element_type=f32)
    d1 = jnp.dot(den_ref[1], erep_ref[...], preferred_element_type=f32)
    a0 = num_ref[0] / (d0 + 1e-16)
    a1 = num_ref[1] / (d1 + 1e-16)
    z = jnp.concatenate([a0, a1], axis=1)
    z = jnp.where(z > 0, z, jnp.exp(z) - 1.0)
    h2 = jnp.dot(z, w2_ref[...], preferred_element_type=f32)
    ss_ref[0] = jnp.dot(h2, a2s_ref[...], preferred_element_type=f32)
    sd_ref[0] = jnp.dot(h2, a2d_ref[...], preferred_element_type=f32)
    h2_ref[0] = jnp.where(c == 0, h2[:, :32], h2[:, 32:])


def _tc2(num1, den1, Erep, W2, A2s, A2d):
    return pl.pallas_call(
        _tc2_body,
        grid=(NC, NB),
        in_specs=[
            pl.BlockSpec((NC, BN, IN_F), lambda c, i: (0, i, 0)),
            pl.BlockSpec((NC, BN, 16), lambda c, i: (0, i, 0)),
            pl.BlockSpec((16, HID // 2), lambda c, i: (0, 0)),
            pl.BlockSpec((HID, C), lambda c, i: (0, 0)),
            pl.BlockSpec((C, 16), lambda c, i: (0, 0)),
            pl.BlockSpec((C, 16), lambda c, i: (0, 0)),
        ],
        out_specs=[
            pl.BlockSpec((1, BN, 32), lambda c, i: (c, i, 0)),
            pl.BlockSpec((1, BN, 16), lambda c, i: (c, i, 0)),
            pl.BlockSpec((1, BN, 16), lambda c, i: (c, i, 0)),
        ],
        out_shape=[
            jax.ShapeDtypeStruct((NC, N, 32), f32),
            jax.ShapeDtypeStruct((NC, N, 16), f32),
            jax.ShapeDtypeStruct((NC, N, 16), f32),
        ],
    )(num1, den1, Erep, W2, A2s, A2d)


def _tc3_body(num_ref, den_ref, e2_ref, out_ref):
    d = jnp.dot(den_ref[0], e2_ref[...], preferred_element_type=f32)
    z = jnp.concatenate([num_ref[0], num_ref[1]], axis=1) / (d + 1e-16)
    z = jnp.where(z > 0, z, jnp.exp(z) - 1.0)
    m = jnp.max(z, axis=1, keepdims=True)
    t = z - m
    out_ref[...] = t - jnp.log(jnp.sum(jnp.exp(t), axis=1, keepdims=True))


def _tc3(num2, den2, E2):
    return pl.pallas_call(
        _tc3_body,
        grid=(NB,),
        in_specs=[
            pl.BlockSpec((NC, BN, 32), lambda i: (0, i, 0)),
            pl.BlockSpec((NC, BN, 16), lambda i: (0, i, 0)),
            pl.BlockSpec((16, C), lambda i: (0, 0)),
        ],
        out_specs=pl.BlockSpec((BN, C), lambda i: (i, 0)),
        out_shape=jax.ShapeDtypeStruct((N, C), f32),
    )(num2, den2, E2)


def _make_sc_edge(F, U):
    n_iter = NCHUNK // U
    n_pairs = F // 32
    mesh = plsc.VectorSubcoreMesh(core_axis_name="c", subcore_axis_name="s")

    def body(h_tab, ss_tab, sd_tab, idxtab, num_out, den_out,
             idxb, rows, ssb, sdb, accn, accd, *sems):
        gsem = sems[:U]
        ssem = sems[U:]
        cid = lax.axis_index("c")
        sid = lax.axis_index("s")
        chunk_base = (cid * NS + sid) * NCHUNK

        zero16 = jnp.zeros((L,), f32)

        def zrow(k, _):
            def zcol(g, _):
                rows[0, k, pl.ds(g * L, L)] = zero16
                return 0
            lax.fori_loop(0, F // L, zcol, 0)
            ssb[0, k, :] = zero16
            return 0
        lax.fori_loop(0, K, zrow, 0)

        nz = jnp.where(sid < 15, NPT // K, NPT_LAST // K)

        def zacc(t, _):
            off = sid * NPT + t * K
            pltpu.sync_copy(rows.at[0], accn.at[pl.ds(off, K)])
            pltpu.sync_copy(ssb.at[0], accd.at[pl.ds(off, K)])
            return 0
        lax.fori_loop(0, nz, zacc, 0)
        plsc.subcore_barrier()

        def giter(t, _):
            g = []
            for p in range(U):
                ch = chunk_base + t * U + p
                pltpu.sync_copy(idxtab.at[ch], idxb.at[p])
                g.append((
                    pltpu.async_copy(h_tab.at[idxb.at[p, 0]], rows.at[p],
                                     gsem[p]),
                    pltpu.async_copy(ss_tab.at[idxb.at[p, 0]], ssb.at[p],
                                     gsem[p]),
                    pltpu.async_copy(sd_tab.at[idxb.at[p, 1]], sdb.at[p],
                                     gsem[p]),
                ))
            s = []
            for p in range(U):
                for d in g[p]:
                    d.wait()

                def scale(k, _):
                    e = ssb[p, k, :] + sdb[p, k, :]
                    w = jnp.exp(jnp.maximum(e, 0.2 * e))
                    ssb[p, k, :] = w
                    for q in range(n_pairs):
                        wv = w.at[jnp.full((L,), q, jnp.int32)].get(
                            mode="promise_in_bounds")
                        for half in range(2):
                            c0 = q * 32 + half * L
                            rows[p, k, pl.ds(c0, L)] = (
                                rows[p, k, pl.ds(c0, L)] * wv)
                    return 0
                lax.fori_loop(0, K, scale, 0)
                s.append(pltpu.async_copy(rows.at[p], accn.at[idxb.at[p, 2]],
                                          ssem[p], add=True))
                s.append(pltpu.async_copy(ssb.at[p], accd.at[idxb.at[p, 2]],
                                          ssem[p], add=True))
            for d in s:
                d.wait()
            return 0
        lax.fori_loop(0, n_iter, giter, 0)
        plsc.subcore_barrier()

        off = sid * NPT

        @pl.when(sid < 15)
        def _():
            pltpu.sync_copy(accn.at[pl.ds(off, NPT)],
                            num_out.at[cid, pl.ds(off, NPT)])
            pltpu.sync_copy(accd.at[pl.ds(off, NPT)],
                            den_out.at[cid, pl.ds(off, NPT)])

        @pl.when(sid == 15)
        def _():
            pltpu.sync_copy(accn.at[pl.ds(off, NPT_LAST)],
                            num_out.at[cid, pl.ds(off, NPT_LAST)])
            pltpu.sync_copy(accd.at[pl.ds(off, NPT_LAST)],
                            den_out.at[cid, pl.ds(off, NPT_LAST)])

    return pl.kernel(
        body,
        out_type=(
            jax.ShapeDtypeStruct((NC, N, F), f32),
            jax.ShapeDtypeStruct((NC, N, 16), f32),
        ),
        mesh=mesh,
        compiler_params=pltpu.CompilerParams(use_tc_tiling_on_sc=False),
        scratch_types=[
            pltpu.VMEM((U, 3, K), jnp.int32),
            pltpu.VMEM((U, K, F), f32),
            pltpu.VMEM((U, K, 16), f32),
            pltpu.VMEM((U, K, 16), f32),
            pltpu.VMEM_SHARED((N, F), f32),
            pltpu.VMEM_SHARED((N, 16), f32),
        ] + [pltpu.SemaphoreType.DMA] * (2 * U),
    )


_sc_edge_128 = _make_sc_edge(IN_F, 2)
_sc_edge_32 = _make_sc_edge(32, 10)


def kernel(x, edge_index, W1, a1_src, a1_dst, W2, a2_src, a2_dst):
    src = edge_index[0]
    dst = edge_index[1]
    base = jnp.stack([src, dst, dst])
    idx_parts = []
    for c in range(NC):
        off = jnp.array([c * N, c * N, 0], jnp.int32)[:, None]
        t = (base + off).reshape(3, NS, NCHUNK, K).transpose(1, 2, 0, 3)
        idx_parts.append(t)
    idxtab = jnp.stack(idx_parts).reshape(NC * NS * NCHUNK, 3, K)

    eye = jnp.eye(16, dtype=f32)
    A1s = (a1_src.reshape(2, 4, DH1)[:, :, :, None]
           * eye[None, :4, None, :]).reshape(2, IN_F, 16)
    A1d = (a1_dst.reshape(2, 4, DH1)[:, :, :, None]
           * eye[None, :4, None, :]).reshape(2, IN_F, 16)
    A2s = jnp.zeros((C, 16), f32).at[:, 0].set(a2_src[0])
    A2d = jnp.zeros((C, 16), f32).at[:, 0].set(a2_dst[0])
    Erep = jnp.repeat(eye[:, :4], DH1, axis=1)
    E2 = jnp.zeros((16, C), f32).at[0, :].set(1.0)

    h1, ss1, sd1 = _tc1(x, W1, A1s, A1d)
    num1, den1 = _sc_edge_128(
        h1.reshape(NC * N, IN_F), ss1.reshape(NC * N, 16),
        sd1.reshape(NC * N, 16), idxtab)
    h2, ss2, sd2 = _tc2(num1, den1, Erep, W2, A2s, A2d)
    num2, den2 = _sc_edge_32(
        h2.reshape(NC * N, 32), ss2.reshape(NC * N, 16),
        sd2.reshape(NC * N, 16), idxtab)
    return _tc3(num2, den2, E2)
